# Initial kernel scaffold; baseline (speedup 1.0000x reference)
import jax
import jax.numpy as jnp
from jax.experimental import pallas as pl


def kernel(node_feats, edge_attr, edge_index, node_encoder, edge_encoder, edge_nets, node_nets, edge_decoder, edge_out):
    raise NotImplementedError("write your pallas kernel here")


# speedup vs baseline: 2.6869x; 2.6869x over previous
import functools

import jax
import jax.numpy as jnp
from jax import lax
from jax.experimental import pallas as pl
from jax.experimental.pallas import tpu as pltpu
from jax.experimental.pallas import tpu_sc as plsc

_N = 10000
_E = 160000
_NH = 256

_NC = 2
_NS = 16
_NW = _NC * _NS

_BN = 2000
_BE = 4000

_GCH = 40
_SCH = 80


def _ln_silu(h):
    m = jnp.mean(h, axis=-1, keepdims=True)
    v = jnp.mean((h - m) ** 2, axis=-1, keepdims=True)
    h = (h - m) * lax.rsqrt(v + 1e-5)
    return h * jax.nn.sigmoid(h)


def _dot(a, w):
    return jnp.dot(a, w, preferred_element_type=jnp.float32)


def _mlp_body(n_in, n_add, acts, *refs):
    k = 0
    xs = refs[k:k + n_in]; k += n_in
    adds = refs[k:k + n_add]; k += n_add
    w1s = refs[k:k + n_in]; k += n_in
    b1 = refs[k]; k += 1
    n_layers = len(acts)
    rest = []
    for _ in range(n_layers - 1):
        rest.append((refs[k], refs[k + 1]))
        k += 2
    out = refs[-1]

    h = _dot(xs[0][...], w1s[0][...])
    for i in range(1, n_in):
        h = h + _dot(xs[i][...], w1s[i][...])
    h = h + b1[...]
    for a in adds:
        h = h + a[...]

    def act(h, a):
        if a == 'ln_silu':
            return _ln_silu(h)
        if a == 'tanh':
            return jnp.tanh(h)
        return h

    h = act(h, acts[0])
    for (w, b), a in zip(rest, acts[1:]):
        h = act(_dot(h, w[...]) + b[...], a)
    out[...] = h


def _row_spec(block_rows, dim):
    return pl.BlockSpec((block_rows, dim), lambda i: (i, 0))


def _full_spec(arr):
    return pl.BlockSpec(arr.shape, lambda i: (0, 0))


def _mlp_call(xs, adds, layers, acts, block_rows, out_dim):
    rows = xs[0].shape[0]
    grid = (rows // block_rows,)
    n_in, n_add = len(xs), len(adds)
    w1s, b1 = layers[0]
    flat_w = list(w1s) + [b1]
    for (w, b) in layers[1:]:
        flat_w += [w, b]
    in_arrays = list(xs) + list(adds) + flat_w
    in_specs = (
        [_row_spec(block_rows, a.shape[1]) for a in xs]
        + [_row_spec(block_rows, a.shape[1]) for a in adds]
        + [_full_spec(w) for w in flat_w]
    )
    body = functools.partial(_mlp_body, n_in, n_add, acts)
    return pl.pallas_call(
        body,
        grid=grid,
        in_specs=in_specs,
        out_specs=_row_spec(block_rows, out_dim),
        out_shape=jax.ShapeDtypeStruct((rows, out_dim), jnp.float32),
    )(*in_arrays)


def _tables_body(x, ix, ws1, ws2, wd1, wd2, gs, gd):
    gs[...] = _dot(x[...], ws1[...]) + _dot(ix[...], ws2[...])
    gd[...] = _dot(x[...], wd1[...]) + _dot(ix[...], wd2[...])


def _tables_call(x, ix, ws1, ws2, wd1, wd2):
    grid = (_N // _BN,)
    spec = _row_spec(_BN, _NH)
    return pl.pallas_call(
        _tables_body,
        grid=grid,
        in_specs=[spec, spec] + [_full_spec(w) for w in (ws1, ws2, wd1, wd2)],
        out_specs=(spec, spec),
        out_shape=(jax.ShapeDtypeStruct((_N, _NH), jnp.float32),
                   jax.ShapeDtypeStruct((_N, _NH), jnp.float32)),
    )(x, ix, ws1, ws2, wd1, wd2)


_EPW = _E // _NW
_GNCH = _EPW // _GCH

_ET = _E // _NS
_SNCH = _ET // _SCH
_NR = 632
_NPAD = _NR * _NS
_HC = _NH // 2


def _sc_mesh():
    return plsc.VectorSubcoreMesh(core_axis_name="c", subcore_axis_name="s")


def _gather_body(gs_h, gd_h, src_h, dst_h, osrc_h, odst_h,
                 idxs_v, idxd_v, rs_v, rd_v, sem_s, sem_d):
    c = lax.axis_index("c")
    s = lax.axis_index("s")
    wid = s * _NC + c
    base = wid * _EPW
    pltpu.sync_copy(src_h.at[pl.ds(base, _EPW)], idxs_v)
    pltpu.sync_copy(dst_h.at[pl.ds(base, _EPW)], idxd_v)

    def step(ci, carry):
        off = ci * _GCH
        a = pltpu.async_copy(gs_h.at[idxs_v.at[pl.ds(off, _GCH)]], rs_v, sem_s)
        b = pltpu.async_copy(gd_h.at[idxd_v.at[pl.ds(off, _GCH)]], rd_v, sem_d)
        a.wait()
        b.wait()
        pltpu.sync_copy(rs_v, osrc_h.at[pl.ds(base + off, _GCH)])
        pltpu.sync_copy(rd_v, odst_h.at[pl.ds(base + off, _GCH)])
        return carry

    lax.fori_loop(0, _GNCH, step, 0)


def _sc_gather(gs, gd, src, dst):
    out_t = (jax.ShapeDtypeStruct((_E, _NH), jnp.float32),
             jax.ShapeDtypeStruct((_E, _NH), jnp.float32))
    scratch = [
        pltpu.VMEM((_EPW,), jnp.int32),
        pltpu.VMEM((_EPW,), jnp.int32),
        pltpu.VMEM((_GCH, _NH), jnp.float32),
        pltpu.VMEM((_GCH, _NH), jnp.float32),
        pltpu.SemaphoreType.DMA,
        pltpu.SemaphoreType.DMA,
    ]
    f = pl.kernel(_gather_body, out_type=out_t, mesh=_sc_mesh(),
                  scratch_types=scratch)
    return f(gs, gd, src, dst)


def _scatter_body(eu_h, dst3_h, src3_h, z_h, sdst_h, ssrc_h,
                  idxd_v, idxs_v, buf, acc_sh):
    c = lax.axis_index("c")
    s = lax.axis_index("s")
    co = c * _HC
    pltpu.sync_copy(dst3_h.at[s], idxd_v)
    pltpu.sync_copy(src3_h.at[s], idxs_v)
    for q in range(2):
        idx_v = idxd_v if q == 0 else idxs_v
        out_h = sdst_h if q == 0 else ssrc_h
        pltpu.sync_copy(z_h, acc_sh.at[pl.ds(s * _NR, _NR)])
        plsc.subcore_barrier()

        def step(ci, carry):
            pltpu.sync_copy(
                eu_h.at[pl.ds(s * _ET + ci * _SCH, _SCH), pl.ds(co, _HC)],
                buf)
            pltpu.sync_copy(buf, acc_sh.at[idx_v.at[ci]], add=True)
            return carry

        lax.fori_loop(0, _SNCH, step, 0)
        plsc.subcore_barrier()
        pltpu.sync_copy(acc_sh.at[pl.ds(s * _NR, _NR)],
                        out_h.at[pl.ds(s * _NR, _NR), pl.ds(co, _HC)])
        plsc.subcore_barrier()


def _sc_scatter(e_upd, dst3, src3, zeros):
    out_t = (jax.ShapeDtypeStruct((_NPAD, _NH), jnp.float32),
             jax.ShapeDtypeStruct((_NPAD, _NH), jnp.float32))
    scratch = [
        pltpu.VMEM((_SNCH, _SCH), jnp.int32),
        pltpu.VMEM((_SNCH, _SCH), jnp.int32),
        pltpu.VMEM((_SCH, _HC), jnp.float32),
        pltpu.VMEM_SHARED((_NPAD, _HC), jnp.float32),
    ]
    f = pl.kernel(_scatter_body, out_type=out_t, mesh=_sc_mesh(),
                  scratch_types=scratch)
    s_dst, s_src = f(e_upd, dst3, src3, zeros)
    return s_dst[:_N], s_src[:_N]


def _prep_layers(params, first=True):
    out = []
    for i, (w, b) in enumerate(params):
        wb = w
        bb = b.reshape(1, -1)
        if i == 0 and first:
            out.append(([wb], bb))
        else:
            out.append((wb, bb))
    return out


_ACTS3 = ('ln_silu', 'ln_silu', 'tanh')


def kernel(node_feats, edge_attr, edge_index, node_encoder, edge_encoder,
           edge_nets, node_nets, edge_decoder, edge_out):
    assert node_feats.shape[0] == _N and edge_attr.shape[0] == _E
    src = edge_index[0]
    dst = edge_index[1]
    dst3 = dst.reshape(_NS, _SNCH, _SCH)
    src3 = src.reshape(_NS, _SNCH, _SCH)
    zeros = jnp.zeros((_NR, _HC), jnp.float32)

    x = _mlp_call([node_feats], [], _prep_layers(node_encoder), _ACTS3,
                  _BN, _NH)
    e = _mlp_call([edge_attr], [], _prep_layers(edge_encoder), _ACTS3,
                  _BE, _NH)
    ix, ie = x, e

    for i in range(len(edge_nets)):
        w1, b1 = edge_nets[i][0]
        w1b = w1
        wa, wb_, ws1, ws2, wd1, wd2 = (w1b[j * _NH:(j + 1) * _NH]
                                       for j in range(6))
        gs, gd = _tables_call(x, ix, ws1, ws2, wd1, wd2)
        g_src, g_dst = _sc_gather(gs, gd, src, dst)
        e_layers = ([([wa, wb_], b1.reshape(1, -1))]
                    + _prep_layers(edge_nets[i][1:], first=False))
        e_upd = _mlp_call([e, ie], [g_src, g_dst], e_layers, _ACTS3,
                          _BE, _NH)
        s_dst, s_src = _sc_scatter(e_upd, dst3, src3, zeros)

        wn1, bn1 = node_nets[i][0]
        wn1b = wn1
        n_parts = [wn1b[j * _NH:(j + 1) * _NH] for j in range(4)]
        n_layers = ([(n_parts, bn1.reshape(1, -1))]
                    + _prep_layers(node_nets[i][1:], first=False))
        x = _mlp_call([s_dst, s_src, x, ix], [], n_layers, _ACTS3, _BN, _NH)
        e = e_upd

    dec_layers = (_prep_layers(edge_decoder)
                  + _prep_layers(edge_out, first=False))
    acts5 = ('ln_silu', 'ln_silu', 'tanh', 'ln_silu', 'none')
    out = _mlp_call([e], [], dec_layers, acts5, _BE, 1)
    return out[:, 0]


# speedup vs baseline: 3.0486x; 1.1346x over previous
import functools

import jax
import jax.numpy as jnp
from jax import lax
from jax.experimental import pallas as pl
from jax.experimental.pallas import tpu as pltpu
from jax.experimental.pallas import tpu_sc as plsc

_N = 10000
_E = 160000
_NH = 256

_NC = 2
_NS = 16
_NW = _NC * _NS

_BN = 2000
_BE = 4000

_GCH = 40
_SCH = 80


def _ln_silu(h):
    m = jnp.mean(h, axis=-1, keepdims=True)
    v = jnp.mean((h - m) ** 2, axis=-1, keepdims=True)
    h = (h - m) * lax.rsqrt(v + 1e-5)
    return h * jax.nn.sigmoid(h)


def _dot(a, w):
    return jnp.dot(a, w, preferred_element_type=jnp.float32)


def _mlp_body(n_in, n_add, acts, *refs):
    k = 0
    xs = refs[k:k + n_in]; k += n_in
    adds = refs[k:k + n_add]; k += n_add
    w1s = refs[k:k + n_in]; k += n_in
    b1 = refs[k]; k += 1
    n_layers = len(acts)
    rest = []
    for _ in range(n_layers - 1):
        rest.append((refs[k], refs[k + 1]))
        k += 2
    out = refs[-1]

    h = _dot(xs[0][...], w1s[0][...])
    for i in range(1, n_in):
        h = h + _dot(xs[i][...], w1s[i][...])
    h = h + b1[...]
    for a in adds:
        h = h + a[...]

    def act(h, a):
        if a == 'ln_silu':
            return _ln_silu(h)
        if a == 'tanh':
            return jnp.tanh(h)
        return h

    h = act(h, acts[0])
    for (w, b), a in zip(rest, acts[1:]):
        h = act(_dot(h, w[...]) + b[...], a)
    out[...] = h


def _row_spec(block_rows, dim):
    return pl.BlockSpec((block_rows, dim), lambda i: (i, 0))


def _full_spec(arr):
    return pl.BlockSpec(arr.shape, lambda i: (0, 0))


def _mlp_call(xs, adds, layers, acts, block_rows, out_dim):
    rows = xs[0].shape[0]
    grid = (rows // block_rows,)
    n_in, n_add = len(xs), len(adds)
    w1s, b1 = layers[0]
    flat_w = list(w1s) + [b1]
    for (w, b) in layers[1:]:
        flat_w += [w, b]
    in_arrays = list(xs) + list(adds) + flat_w
    in_specs = (
        [_row_spec(block_rows, a.shape[1]) for a in xs]
        + [_row_spec(block_rows, a.shape[1]) for a in adds]
        + [_full_spec(w) for w in flat_w]
    )
    body = functools.partial(_mlp_body, n_in, n_add, acts)
    return pl.pallas_call(
        body,
        grid=grid,
        in_specs=in_specs,
        out_specs=_row_spec(block_rows, out_dim),
        out_shape=jax.ShapeDtypeStruct((rows, out_dim), jnp.float32),
    )(*in_arrays)


def _tables_body(x, ix, ws1, ws2, wd1, wd2, gs, gd):
    gs[...] = _dot(x[...], ws1[...]) + _dot(ix[...], ws2[...])
    gd[...] = _dot(x[...], wd1[...]) + _dot(ix[...], wd2[...])


def _tables_call(x, ix, ws1, ws2, wd1, wd2):
    grid = (_N // _BN,)
    spec = _row_spec(_BN, _NH)
    return pl.pallas_call(
        _tables_body,
        grid=grid,
        in_specs=[spec, spec] + [_full_spec(w) for w in (ws1, ws2, wd1, wd2)],
        out_specs=(spec, spec),
        out_shape=(jax.ShapeDtypeStruct((_N, _NH), jnp.float32),
                   jax.ShapeDtypeStruct((_N, _NH), jnp.float32)),
    )(x, ix, ws1, ws2, wd1, wd2)


_EPW = _E // _NW
_GNCH = _EPW // _GCH

_ET = _E // _NS
_SNCH = _ET // _SCH
_NR = 632
_NPAD = _NR * _NS
_HC = _NH // 2


def _sc_mesh():
    return plsc.VectorSubcoreMesh(core_axis_name="c", subcore_axis_name="s")


_GNB = 5


def _gather_body(gs_h, gd_h, src_h, dst_h, osrc_h, odst_h,
                 idxs_v, idxd_v, rs_v, rd_v, gsem, wsem):
    c = lax.axis_index("c")
    s = lax.axis_index("s")
    wid = s * _NC + c
    base = wid * _EPW
    pltpu.sync_copy(src_h.at[pl.ds(base, _EPW)], idxs_v)
    pltpu.sync_copy(dst_h.at[pl.ds(base, _EPW)], idxd_v)

    def outer(i, carry):
        cps = []
        for b in range(_GNB):
            off = (i * _GNB + b) * _GCH
            cps.append(pltpu.async_copy(
                gs_h.at[idxs_v.at[pl.ds(off, _GCH)]], rs_v.at[b], gsem))
            cps.append(pltpu.async_copy(
                gd_h.at[idxd_v.at[pl.ds(off, _GCH)]], rd_v.at[b], gsem))
        for cp in cps:
            cp.wait()
        wps = []
        for b in range(_GNB):
            off = (i * _GNB + b) * _GCH
            wps.append(pltpu.async_copy(
                rs_v.at[b], osrc_h.at[pl.ds(base + off, _GCH)], wsem))
            wps.append(pltpu.async_copy(
                rd_v.at[b], odst_h.at[pl.ds(base + off, _GCH)], wsem))
        for wp in wps:
            wp.wait()
        return carry

    lax.fori_loop(0, _GNCH // _GNB, outer, 0)


def _sc_gather(gs, gd, src, dst):
    out_t = (jax.ShapeDtypeStruct((_E, _NH), jnp.float32),
             jax.ShapeDtypeStruct((_E, _NH), jnp.float32))
    scratch = [
        pltpu.VMEM((_EPW,), jnp.int32),
        pltpu.VMEM((_EPW,), jnp.int32),
        pltpu.VMEM((_GNB, _GCH, _NH), jnp.float32),
        pltpu.VMEM((_GNB, _GCH, _NH), jnp.float32),
        pltpu.SemaphoreType.DMA,
        pltpu.SemaphoreType.DMA,
    ]
    f = pl.kernel(_gather_body, out_type=out_t, mesh=_sc_mesh(),
                  scratch_types=scratch)
    return f(gs, gd, src, dst)


_SNB = 2


def _scatter_body(eu_h, dst3_h, src3_h, z_h, sdst_h, ssrc_h,
                  idx_v, buf, acc_sh, lsem, asem):
    c = lax.axis_index("c")
    s = lax.axis_index("s")
    co = c * _HC
    for q in range(2):
        in3_h = dst3_h if q == 0 else src3_h
        out_h = sdst_h if q == 0 else ssrc_h
        pltpu.sync_copy(in3_h.at[s], idx_v)
        pltpu.sync_copy(z_h, acc_sh.at[pl.ds(s * _NR, _NR)])
        plsc.subcore_barrier()

        def outer(i, carry):
            cps = []
            for b in range(_SNB):
                ci = i * _SNB + b
                cps.append(pltpu.async_copy(
                    eu_h.at[pl.ds(s * _ET + ci * _SCH, _SCH),
                            pl.ds(co, _HC)],
                    buf.at[b], lsem))
            for cp in cps:
                cp.wait()
            aps = []
            for b in range(_SNB):
                ci = i * _SNB + b
                aps.append(pltpu.async_copy(
                    buf.at[b], acc_sh.at[idx_v.at[ci]], asem, add=True))
            for ap in aps:
                ap.wait()
            return carry

        n_main = (_SNCH // _SNB) * _SNB
        lax.fori_loop(0, _SNCH // _SNB, outer, 0)

        for k, ci in enumerate(range(n_main, _SNCH)):
            b = k % _SNB
            pltpu.sync_copy(
                eu_h.at[pl.ds(s * _ET + ci * _SCH, _SCH), pl.ds(co, _HC)],
                buf.at[b])
            pltpu.sync_copy(buf.at[b], acc_sh.at[idx_v.at[ci]], add=True)

        plsc.subcore_barrier()
        pltpu.sync_copy(acc_sh.at[pl.ds(s * _NR, _NR)],
                        out_h.at[pl.ds(s * _NR, _NR), pl.ds(co, _HC)])
        plsc.subcore_barrier()


def _sc_scatter(e_upd, dst3, src3, zeros):
    out_t = (jax.ShapeDtypeStruct((_NPAD, _NH), jnp.float32),
             jax.ShapeDtypeStruct((_NPAD, _NH), jnp.float32))
    scratch = [
        pltpu.VMEM((_SNCH, _SCH), jnp.int32),
        pltpu.VMEM((_SNB, _SCH, _HC), jnp.float32),
        pltpu.VMEM_SHARED((_NPAD, _HC), jnp.float32),
        pltpu.SemaphoreType.DMA,
        pltpu.SemaphoreType.DMA,
    ]
    f = pl.kernel(_scatter_body, out_type=out_t, mesh=_sc_mesh(),
                  scratch_types=scratch)
    s_dst, s_src = f(e_upd, dst3, src3, zeros)
    return s_dst[:_N], s_src[:_N]


def _prep_layers(params, first=True):
    out = []
    for i, (w, b) in enumerate(params):
        wb = w
        bb = b.reshape(1, -1)
        if i == 0 and first:
            out.append(([wb], bb))
        else:
            out.append((wb, bb))
    return out


_ACTS3 = ('ln_silu', 'ln_silu', 'tanh')


def kernel(node_feats, edge_attr, edge_index, node_encoder, edge_encoder,
           edge_nets, node_nets, edge_decoder, edge_out):
    assert node_feats.shape[0] == _N and edge_attr.shape[0] == _E
    src = edge_index[0]
    dst = edge_index[1]
    dst3 = dst.reshape(_NS, _SNCH, _SCH)
    src3 = src.reshape(_NS, _SNCH, _SCH)
    zeros = jnp.zeros((_NR, _HC), jnp.float32)

    x = _mlp_call([node_feats], [], _prep_layers(node_encoder), _ACTS3,
                  _BN, _NH)
    e = _mlp_call([edge_attr], [], _prep_layers(edge_encoder), _ACTS3,
                  _BE, _NH)
    ix, ie = x, e

    for i in range(len(edge_nets)):
        w1, b1 = edge_nets[i][0]
        w1b = w1
        wa, wb_, ws1, ws2, wd1, wd2 = (w1b[j * _NH:(j + 1) * _NH]
                                       for j in range(6))
        gs, gd = _tables_call(x, ix, ws1, ws2, wd1, wd2)
        g_src, g_dst = _sc_gather(gs, gd, src, dst)
        e_layers = ([([wa, wb_], b1.reshape(1, -1))]
                    + _prep_layers(edge_nets[i][1:], first=False))
        e_upd = _mlp_call([e, ie], [g_src, g_dst], e_layers, _ACTS3,
                          _BE, _NH)
        s_dst, s_src = _sc_scatter(e_upd, dst3, src3, zeros)

        wn1, bn1 = node_nets[i][0]
        wn1b = wn1
        n_parts = [wn1b[j * _NH:(j + 1) * _NH] for j in range(4)]
        n_layers = ([(n_parts, bn1.reshape(1, -1))]
                    + _prep_layers(node_nets[i][1:], first=False))
        x = _mlp_call([s_dst, s_src, x, ix], [], n_layers, _ACTS3, _BN, _NH)
        e = e_upd

    dec_layers = (_prep_layers(edge_decoder)
                  + _prep_layers(edge_out, first=False))
    acts5 = ('ln_silu', 'ln_silu', 'tanh', 'ln_silu', 'none')
    out = _mlp_call([e], [], dec_layers, acts5, _BE, 1)
    return out[:, 0]


# speedup vs baseline: 3.1904x; 1.0465x over previous
import functools

import jax
import jax.numpy as jnp
from jax import lax
from jax.experimental import pallas as pl
from jax.experimental.pallas import tpu as pltpu
from jax.experimental.pallas import tpu_sc as plsc

_N = 10000
_E = 160000
_NH = 256

_NC = 2
_NS = 16
_NW = _NC * _NS

_BN = 2000
_BE = 3200

_EA = 76800
_EB = _E - _EA

_GCH = 40
_SCH = 80


def _ln_silu(h):
    m = jnp.mean(h, axis=-1, keepdims=True)
    v = jnp.mean((h - m) ** 2, axis=-1, keepdims=True)
    h = (h - m) * lax.rsqrt(v + 1e-5)
    return h * jax.nn.sigmoid(h)


def _dot(a, w):
    return jnp.dot(a, w, preferred_element_type=jnp.float32)


def _mlp_body(n_in, n_add, acts, *refs):
    k = 0
    xs = refs[k:k + n_in]; k += n_in
    adds = refs[k:k + n_add]; k += n_add
    w1s = refs[k:k + n_in]; k += n_in
    b1 = refs[k]; k += 1
    n_layers = len(acts)
    rest = []
    for _ in range(n_layers - 1):
        rest.append((refs[k], refs[k + 1]))
        k += 2
    out = refs[-1]

    h = _dot(xs[0][...], w1s[0][...])
    for i in range(1, n_in):
        h = h + _dot(xs[i][...], w1s[i][...])
    h = h + b1[...]
    for a in adds:
        h = h + a[...]

    def act(h, a):
        if a == 'ln_silu':
            return _ln_silu(h)
        if a == 'tanh':
            return jnp.tanh(h)
        return h

    h = act(h, acts[0])
    for (w, b), a in zip(rest, acts[1:]):
        h = act(_dot(h, w[...]) + b[...], a)
    out[...] = h


def _row_spec(block_rows, dim):
    return pl.BlockSpec((block_rows, dim), lambda i: (i, 0))


def _full_spec(arr):
    return pl.BlockSpec(arr.shape, lambda i: (0, 0))


def _mlp_call(xs, adds, layers, acts, block_rows, out_dim):
    rows = xs[0].shape[0]
    grid = (rows // block_rows,)
    n_in, n_add = len(xs), len(adds)
    w1s, b1 = layers[0]
    flat_w = list(w1s) + [b1]
    for (w, b) in layers[1:]:
        flat_w += [w, b]
    in_arrays = list(xs) + list(adds) + flat_w
    in_specs = (
        [_row_spec(block_rows, a.shape[1]) for a in xs]
        + [_row_spec(block_rows, a.shape[1]) for a in adds]
        + [_full_spec(w) for w in flat_w]
    )
    body = functools.partial(_mlp_body, n_in, n_add, acts)
    return pl.pallas_call(
        body,
        grid=grid,
        in_specs=in_specs,
        out_specs=_row_spec(block_rows, out_dim),
        out_shape=jax.ShapeDtypeStruct((rows, out_dim), jnp.float32),
    )(*in_arrays)


def _tables_body(x, ix, ws1, ws2, wd1, wd2, gs, gd):
    gs[...] = _dot(x[...], ws1[...]) + _dot(ix[...], ws2[...])
    gd[...] = _dot(x[...], wd1[...]) + _dot(ix[...], wd2[...])


def _tables_call(x, ix, ws1, ws2, wd1, wd2):
    grid = (_N // _BN,)
    spec = _row_spec(_BN, _NH)
    return pl.pallas_call(
        _tables_body,
        grid=grid,
        in_specs=[spec, spec] + [_full_spec(w) for w in (ws1, ws2, wd1, wd2)],
        out_specs=(spec, spec),
        out_shape=(jax.ShapeDtypeStruct((_N, _NH), jnp.float32),
                   jax.ShapeDtypeStruct((_N, _NH), jnp.float32)),
    )(x, ix, ws1, ws2, wd1, wd2)


_NR = 632
_NPAD = _NR * _NS
_HC = _NH // 2


def _sc_mesh():
    return plsc.VectorSubcoreMesh(core_axis_name="c", subcore_axis_name="s")


_GNB = 5


def _gather_body(epw, nch, gs_h, gd_h, src_h, dst_h, osrc_h, odst_h,
                 idxs_v, idxd_v, rs_v, rd_v, gsem, wsem):
    c = lax.axis_index("c")
    s = lax.axis_index("s")
    wid = s * _NC + c
    base = wid * epw
    pltpu.sync_copy(src_h.at[pl.ds(base, epw)], idxs_v)
    pltpu.sync_copy(dst_h.at[pl.ds(base, epw)], idxd_v)

    def outer(i, carry):
        cps = []
        for b in range(_GNB):
            off = (i * _GNB + b) * _GCH
            cps.append(pltpu.async_copy(
                gs_h.at[idxs_v.at[pl.ds(off, _GCH)]], rs_v.at[b], gsem))
            cps.append(pltpu.async_copy(
                gd_h.at[idxd_v.at[pl.ds(off, _GCH)]], rd_v.at[b], gsem))
        for cp in cps:
            cp.wait()
        wps = []
        for b in range(_GNB):
            off = (i * _GNB + b) * _GCH
            wps.append(pltpu.async_copy(
                rs_v.at[b], osrc_h.at[pl.ds(base + off, _GCH)], wsem))
            wps.append(pltpu.async_copy(
                rd_v.at[b], odst_h.at[pl.ds(base + off, _GCH)], wsem))
        for wp in wps:
            wp.wait()
        return carry

    lax.fori_loop(0, nch // _GNB, outer, 0)


def _sc_gather(gs, gd, src, dst):
    e_part = src.shape[0]
    epw = e_part // _NW
    nch = epw // _GCH
    out_t = (jax.ShapeDtypeStruct((e_part, _NH), jnp.float32),
             jax.ShapeDtypeStruct((e_part, _NH), jnp.float32))
    scratch = [
        pltpu.VMEM((epw,), jnp.int32),
        pltpu.VMEM((epw,), jnp.int32),
        pltpu.VMEM((_GNB, _GCH, _NH), jnp.float32),
        pltpu.VMEM((_GNB, _GCH, _NH), jnp.float32),
        pltpu.SemaphoreType.DMA,
        pltpu.SemaphoreType.DMA,
    ]
    body = functools.partial(_gather_body, epw, nch)
    f = pl.kernel(body, out_type=out_t, mesh=_sc_mesh(),
                  scratch_types=scratch)
    return f(gs, gd, src, dst)


_SNB = 2


def _scatter_body(et, snch, from_prev, eu_h, dst3_h, src3_h, pd_h, ps_h,
                  sdst_h, ssrc_h, idx_v, buf, acc_sh, lsem, asem):
    c = lax.axis_index("c")
    s = lax.axis_index("s")
    co = c * _HC
    for q in range(2):
        in3_h = dst3_h if q == 0 else src3_h
        out_h = sdst_h if q == 0 else ssrc_h
        prev_h = pd_h if q == 0 else ps_h
        pltpu.sync_copy(in3_h.at[s], idx_v)
        if from_prev:
            pltpu.sync_copy(prev_h.at[pl.ds(s * _NR, _NR), pl.ds(co, _HC)],
                            acc_sh.at[pl.ds(s * _NR, _NR)])
        else:
            pltpu.sync_copy(prev_h, acc_sh.at[pl.ds(s * _NR, _NR)])
        plsc.subcore_barrier()

        def outer(i, carry):
            cps = []
            for b in range(_SNB):
                ci = i * _SNB + b
                cps.append(pltpu.async_copy(
                    eu_h.at[pl.ds(s * et + ci * _SCH, _SCH),
                            pl.ds(co, _HC)],
                    buf.at[b], lsem))
            for cp in cps:
                cp.wait()
            aps = []
            for b in range(_SNB):
                ci = i * _SNB + b
                aps.append(pltpu.async_copy(
                    buf.at[b], acc_sh.at[idx_v.at[ci]], asem, add=True))
            for ap in aps:
                ap.wait()
            return carry

        n_main = (snch // _SNB) * _SNB
        lax.fori_loop(0, snch // _SNB, outer, 0)

        for k, ci in enumerate(range(n_main, snch)):
            b = k % _SNB
            pltpu.sync_copy(
                eu_h.at[pl.ds(s * et + ci * _SCH, _SCH), pl.ds(co, _HC)],
                buf.at[b])
            pltpu.sync_copy(buf.at[b], acc_sh.at[idx_v.at[ci]], add=True)

        plsc.subcore_barrier()
        pltpu.sync_copy(acc_sh.at[pl.ds(s * _NR, _NR)],
                        out_h.at[pl.ds(s * _NR, _NR), pl.ds(co, _HC)])
        plsc.subcore_barrier()


def _sc_scatter(e_upd, dst3, src3, prev_dst, prev_src, from_prev):
    et = e_upd.shape[0] // _NS
    snch = et // _SCH
    out_t = (jax.ShapeDtypeStruct((_NPAD, _NH), jnp.float32),
             jax.ShapeDtypeStruct((_NPAD, _NH), jnp.float32))
    scratch = [
        pltpu.VMEM((snch, _SCH), jnp.int32),
        pltpu.VMEM((_SNB, _SCH, _HC), jnp.float32),
        pltpu.VMEM_SHARED((_NPAD, _HC), jnp.float32),
        pltpu.SemaphoreType.DMA,
        pltpu.SemaphoreType.DMA,
    ]
    body = functools.partial(_scatter_body, et, snch, from_prev)
    f = pl.kernel(body, out_type=out_t, mesh=_sc_mesh(),
                  scratch_types=scratch)
    return f(e_upd, dst3, src3, prev_dst, prev_src)


def _prep_layers(params, first=True):
    out = []
    for i, (w, b) in enumerate(params):
        wb = w
        bb = b.reshape(1, -1)
        if i == 0 and first:
            out.append(([wb], bb))
        else:
            out.append((wb, bb))
    return out


_ACTS3 = ('ln_silu', 'ln_silu', 'tanh')


def kernel(node_feats, edge_attr, edge_index, node_encoder, edge_encoder,
           edge_nets, node_nets, edge_decoder, edge_out):
    assert node_feats.shape[0] == _N and edge_attr.shape[0] == _E
    src = edge_index[0]
    dst = edge_index[1]
    srcA, srcB = src[:_EA], src[_EA:]
    dstA, dstB = dst[:_EA], dst[_EA:]
    dst3A = dstA.reshape(_NS, (_EA // _NS) // _SCH, _SCH)
    src3A = srcA.reshape(_NS, (_EA // _NS) // _SCH, _SCH)
    dst3B = dstB.reshape(_NS, (_EB // _NS) // _SCH, _SCH)
    src3B = srcB.reshape(_NS, (_EB // _NS) // _SCH, _SCH)
    zeros = jnp.zeros((_NR, _HC), jnp.float32)

    x = _mlp_call([node_feats], [], _prep_layers(node_encoder), _ACTS3,
                  _BN, _NH)
    eA = _mlp_call([edge_attr[:_EA]], [], _prep_layers(edge_encoder),
                   _ACTS3, _BE, _NH)
    eB = _mlp_call([edge_attr[_EA:]], [], _prep_layers(edge_encoder),
                   _ACTS3, _BE, _NH)
    ix, ieA, ieB = x, eA, eB

    for i in range(len(edge_nets)):
        w1, b1 = edge_nets[i][0]
        wa, wb_, ws1, ws2, wd1, wd2 = (w1[j * _NH:(j + 1) * _NH]
                                       for j in range(6))
        gs, gd = _tables_call(x, ix, ws1, ws2, wd1, wd2)
        g_srcA, g_dstA = _sc_gather(gs, gd, srcA, dstA)
        g_srcB, g_dstB = _sc_gather(gs, gd, srcB, dstB)
        e_layers = ([([wa, wb_], b1.reshape(1, -1))]
                    + _prep_layers(edge_nets[i][1:], first=False))
        e_updA = _mlp_call([eA, ieA], [g_srcA, g_dstA], e_layers, _ACTS3,
                           _BE, _NH)
        s_dstA, s_srcA = _sc_scatter(e_updA, dst3A, src3A, zeros, zeros,
                                     False)
        e_updB = _mlp_call([eB, ieB], [g_srcB, g_dstB], e_layers, _ACTS3,
                           _BE, _NH)
        s_dst, s_src = _sc_scatter(e_updB, dst3B, src3B, s_dstA, s_srcA,
                                   True)

        wn1, bn1 = node_nets[i][0]
        n_parts = [wn1[j * _NH:(j + 1) * _NH] for j in range(4)]
        n_layers = ([(n_parts, bn1.reshape(1, -1))]
                    + _prep_layers(node_nets[i][1:], first=False))
        x = _mlp_call([s_dst[:_N], s_src[:_N], x, ix], [], n_layers,
                      _ACTS3, _BN, _NH)
        eA, eB = e_updA, e_updB

    dec_layers = (_prep_layers(edge_decoder)
                  + _prep_layers(edge_out, first=False))
    acts5 = ('ln_silu', 'ln_silu', 'tanh', 'ln_silu', 'none')
    outA = _mlp_call([eA], [], dec_layers, acts5, _BE, 1)
    outB = _mlp_call([eB], [], dec_layers, acts5, _BE, 1)
    return jnp.concatenate([outA[:, 0], outB[:, 0]])
